# baseline (speedup 1.0000x reference)
import functools

import jax
import jax.numpy as jnp
from jax import lax
from jax.experimental import pallas as pl
from jax.experimental.pallas import tpu as pltpu
from jax.experimental.pallas import tpu_sc as plsc

NN = 10000
NPAD = 10240
EDG = 160000
FEAT = 256
HALF = 128
NSLAB = 4
ROW_BLK = 1000
NBLK = NN // ROW_BLK

DW = 32
DPW = 5008
DCHUNK = DPW // 16

K = 80
CHI = 16
CHO = 8
PPB = CHI // 4
NP = CHO * PPB
PERT = CHO * CHI * K
STRIPE = 624

_SC_PARAMS = pltpu.CompilerParams(needs_layout_passes=False)


def _deg_call(dst_r, ew_r, zero_hist):
    mesh = plsc.VectorSubcoreMesh(core_axis_name="c", subcore_axis_name="s")

    @functools.partial(
        pl.kernel,
        out_type=jax.ShapeDtypeStruct((DW, NPAD), jnp.float32),
        mesh=mesh,
        compiler_params=_SC_PARAMS,
        scratch_types=[
            pltpu.VMEM((DPW,), jnp.int32),
            pltpu.VMEM((DPW,), jnp.float32),
            pltpu.VMEM((NPAD,), jnp.float32),
        ],
    )
    def body(dst_hbm, ew_hbm, zero_hbm, out_hbm, dst_v, ew_v, hist_v):
        c = lax.axis_index("c")
        s = lax.axis_index("s")
        w = s * 2 + c
        pltpu.sync_copy(dst_hbm.at[w], dst_v)
        pltpu.sync_copy(ew_hbm.at[w], ew_v)
        pltpu.sync_copy(zero_hbm, hist_v)

        def step(i, carry):
            d = dst_v[pl.ds(i * 16, 16)]
            e = ew_v[pl.ds(i * 16, 16)]
            plsc.addupdate_scatter(hist_v, [d], e)
            return carry

        lax.fori_loop(0, DCHUNK, step, 0)
        pltpu.sync_copy(hist_v, out_hbm.at[w])

    return body(dst_r, ew_r, zero_hist)


def _msg_call(xwp, src3, dst3, ew3):
    mesh = plsc.VectorSubcoreMesh(core_axis_name="c", subcore_axis_name="s")

    @functools.partial(
        pl.kernel,
        out_type=jax.ShapeDtypeStruct((2, NSLAB, NN, HALF), jnp.float32),
        mesh=mesh,
        compiler_params=_SC_PARAMS,
        scratch_types=[
            pltpu.VMEM((CHI, K), jnp.int32),
            pltpu.VMEM((CHI, K), jnp.int32),
            pltpu.VMEM((CHI, K), jnp.float32),
            pltpu.VMEM((K, HALF), jnp.float32),
            pltpu.VMEM((K, HALF), jnp.float32),
            pltpu.VMEM((K, HALF), jnp.float32),
            pltpu.VMEM((K, HALF), jnp.float32),
            pltpu.VMEM((16, HALF), jnp.float32),
            pltpu.VMEM_SHARED((NN, HALF), jnp.float32),
            pltpu.SemaphoreType.DMA,
            pltpu.SemaphoreType.DMA,
            pltpu.SemaphoreType.DMA,
            pltpu.SemaphoreType.DMA,
            pltpu.SemaphoreType.DMA,
            pltpu.SemaphoreType.DMA,
            pltpu.SemaphoreType.DMA,
            pltpu.SemaphoreType.DMA,
            pltpu.SemaphoreType.DMA,
        ],
    )
    def body(xwp_hbm, src_hbm, dst_hbm, ew_hbm, out_hbm,
             src_v, dst_v, ew_v, r0, r1, r2, r3, zbuf_v, acc_sh,
             g0, g1, g2, g3, s0, s1, s2, s3, zsem):
        c = lax.axis_index("c")
        s = lax.axis_index("s")
        base = s * STRIPE
        rows = (r0, r1, r2, r3)
        gsem = (g0, g1, g2, g3)
        ssem = (s0, s1, s2, s3)

        zeros16 = jnp.zeros((16,), jnp.float32)

        def zrow(r, carry):
            for q in range(8):
                zbuf_v[r, pl.ds(q * 16, 16)] = zeros16
            return carry

        lax.fori_loop(0, 16, zrow, 0)

        def drain_scatter(i):
            pltpu.make_async_copy(
                xwp_hbm.at[0, 0, pl.ds(0, K)], rows[i], ssem[i]).wait()

        def scale(buf, cc):
            def scale_g(g, carry2):
                wv = ew_v[cc, pl.ds(g * 16, 16)]
                for rr in range(16):
                    w = wv[rr]
                    r = g * 16 + rr
                    for q in range(8):
                        sl = pl.ds(q * 16, 16)
                        buf[r, sl] = buf[r, sl] * w
                return carry2

            lax.fori_loop(0, K // 16, scale_g, 0)

        def slab(j, carry):
            def zloop(t, carry2):
                pltpu.async_copy(
                    zbuf_v, acc_sh.at[pl.ds(base + t * 16, 16)], zsem)
                return carry2

            lax.fori_loop(0, STRIPE // 16, zloop, 0)

            @pl.when(s == 15)
            def _():
                pltpu.async_copy(
                    zbuf_v, acc_sh.at[pl.ds(16 * STRIPE, 16)], zsem)

            def zdrain(t, carry2):
                pltpu.make_async_copy(
                    xwp_hbm.at[0, 0, pl.ds(0, 16)], zbuf_v, zsem).wait()
                return carry2

            lax.fori_loop(0, STRIPE // 16, zdrain, 0)

            @pl.when(s == 15)
            def _():
                pltpu.make_async_copy(
                    xwp_hbm.at[0, 0, pl.ds(0, 16)], zbuf_v, zsem).wait()

            plsc.subcore_barrier()

            def pair(p, carry2):
                is_stage = (p % PPB) == 0
                bo = p // PPB

                @pl.when(jnp.logical_and(is_stage, p > 0))
                def _():
                    for i in range(4):
                        drain_scatter(i)

                @pl.when(is_stage)
                def _():
                    da = pltpu.async_copy(src_hbm.at[s, bo], src_v, zsem)
                    db = pltpu.async_copy(dst_hbm.at[s, bo], dst_v, zsem)
                    dc = pltpu.async_copy(ew_hbm.at[s, bo], ew_v, zsem)
                    da.wait()
                    db.wait()
                    dc.wait()

                @pl.when(jnp.logical_and(jnp.logical_not(is_stage), p > 0))
                def _():
                    for i in range(4):
                        drain_scatter(i)

                q0 = (p % PPB) * 4
                gd = []
                for i in range(4):
                    gd.append(pltpu.async_copy(
                        xwp_hbm.at[c, j].at[src_v.at[q0 + i]],
                        rows[i], gsem[i]))
                for i in range(4):
                    gd[i].wait()
                    scale(rows[i], q0 + i)
                    pltpu.async_copy(
                        rows[i], acc_sh.at[dst_v.at[q0 + i]], ssem[i],
                        add=True)
                return carry2

            lax.fori_loop(0, NP, pair, 0)
            for i in range(4):
                drain_scatter(i)
            plsc.subcore_barrier()
            pltpu.sync_copy(acc_sh.at[pl.ds(base, STRIPE)],
                            out_hbm.at[c, j, pl.ds(base, STRIPE)])

            @pl.when(s == 15)
            def _():
                pltpu.sync_copy(
                    acc_sh.at[pl.ds(16 * STRIPE, NN - 16 * STRIPE)],
                    out_hbm.at[c, j, pl.ds(16 * STRIPE, NN - 16 * STRIPE)])

            plsc.subcore_barrier()
            return carry

        lax.fori_loop(0, NSLAB, slab, 0)

    return body(xwp, src3, dst3, ew3)


def _xw_body(x_ref, w_ref, deg_ref, out_ref):
    deg = jnp.sum(deg_ref[0], axis=0) + 1.0
    dinv = lax.rsqrt(deg)
    xw = jnp.dot(x_ref[0], w_ref[...], preferred_element_type=jnp.float32)
    out_ref[0, 0] = xw * dinv[:, None]


def _xw_call(x4, w, deg3):
    grid = (2, NSLAB, NBLK)
    return pl.pallas_call(
        _xw_body,
        grid=grid,
        in_specs=[
            pl.BlockSpec((1, ROW_BLK, FEAT), lambda c, j, n: (j, n, 0)),
            pl.BlockSpec((FEAT, HALF), lambda c, j, n: (0, c)),
            pl.BlockSpec((1, DW, ROW_BLK), lambda c, j, n: (n, 0, 0)),
        ],
        out_specs=pl.BlockSpec((1, 1, ROW_BLK, HALF), lambda c, j, n: (c, j, n, 0)),
        out_shape=jax.ShapeDtypeStruct((2, NSLAB, NN, HALF), jnp.float32),
    )(x4, w, deg3)


def _rnn_body(s_ref, xwp_ref, deg_ref, b_ref, wih_t_ref, whh_t_ref,
              bih_ref, bhh_ref, out_ref):
    deg = jnp.sum(deg_ref[0], axis=0) + 1.0
    dinv = lax.rsqrt(deg)[:, None]
    b = b_ref[...]
    bias = bih_ref[...] + bhh_ref[...]
    wih_t = wih_t_ref[...]
    whh_t = whh_t_ref[...]

    def conv(j):
        lo = dinv * (s_ref[0, j] + xwp_ref[0, j])
        hi = dinv * (s_ref[1, j] + xwp_ref[1, j])
        return jnp.concatenate([lo, hi], axis=1) + b

    h = jnp.zeros((ROW_BLK, FEAT), dtype=jnp.float32)
    for k, j in enumerate((0, 2, 1, 3)):
        h = jnp.tanh(
            jnp.dot(conv(j), wih_t, preferred_element_type=jnp.float32)
            + jnp.dot(h, whh_t, preferred_element_type=jnp.float32)
            + bias
        )
        if k == 2:
            out_ref[0] = h
    out_ref[1] = h


def _rnn_call(s, xwp, deg3, b, wih_t, whh_t, bih, bhh):
    grid = (NBLK,)
    return pl.pallas_call(
        _rnn_body,
        grid=grid,
        in_specs=[
            pl.BlockSpec((2, NSLAB, ROW_BLK, HALF), lambda n: (0, 0, n, 0)),
            pl.BlockSpec((2, NSLAB, ROW_BLK, HALF), lambda n: (0, 0, n, 0)),
            pl.BlockSpec((1, DW, ROW_BLK), lambda n: (n, 0, 0)),
            pl.BlockSpec((FEAT,), lambda n: (0,)),
            pl.BlockSpec((FEAT, FEAT), lambda n: (0, 0)),
            pl.BlockSpec((FEAT, FEAT), lambda n: (0, 0)),
            pl.BlockSpec((FEAT,), lambda n: (0,)),
            pl.BlockSpec((FEAT,), lambda n: (0,)),
        ],
        out_specs=pl.BlockSpec((2, ROW_BLK, FEAT), lambda n: (0, n, 0)),
        out_shape=jax.ShapeDtypeStruct((2, NN, FEAT), jnp.float32),
    )(s, xwp, deg3, b, wih_t, whh_t, bih, bhh)


def kernel(x_in, edge_index, edge_weight, W, b, W_ih, W_hh, b_ih, b_hh):
    B, T, N, F = x_in.shape
    src = edge_index[0]
    dst = edge_index[1]

    dpad = DW * DPW - EDG
    dst_r = jnp.pad(dst, (0, dpad)).reshape(DW, DPW)
    ew_r = jnp.pad(edge_weight, (0, dpad)).reshape(DW, DPW)
    zero_hist = jnp.zeros((NPAD,), jnp.float32)
    deg_parts = _deg_call(dst_r, ew_r, zero_hist)
    deg3 = (deg_parts[:, :NN]
            .reshape(DW, NBLK, ROW_BLK).transpose(1, 0, 2))

    x4 = x_in.reshape(B * T, N, F)
    xwp = _xw_call(x4, W, deg3)

    mpad = 16 * PERT - EDG
    src3 = jnp.pad(src, (0, mpad)).reshape(16, CHO, CHI, K)
    dst3 = jnp.pad(dst, (0, mpad)).reshape(16, CHO, CHI, K)
    ew3 = jnp.pad(edge_weight, (0, mpad)).reshape(16, CHO, CHI, K)
    s = _msg_call(xwp, src3, dst3, ew3)

    return _rnn_call(s, xwp, deg3, b, W_ih.T, W_hh.T, b_ih, b_hh)


# speedup vs baseline: 1.4922x; 1.4922x over previous
import functools

import jax
import jax.numpy as jnp
from jax import lax
from jax.experimental import pallas as pl
from jax.experimental.pallas import tpu as pltpu
from jax.experimental.pallas import tpu_sc as plsc

NN = 10000
NPAD = 10240
EDG = 160000
FEAT = 256
HALF = 128
NSLAB = 4
ROW_BLK = 1000
NBLK = NN // ROW_BLK

DW = 32
DPW = 5008
DCHUNK = DPW // 16

K = 96
CHI = 15
CHO = 7
PPB = CHI // 3
NP = CHO * PPB
PERT = CHO * CHI * K
STRIPE = 624

_SC_PARAMS = pltpu.CompilerParams(needs_layout_passes=False)


def _deg_call(dst_r, ew_r, zero_hist):
    mesh = plsc.VectorSubcoreMesh(core_axis_name="c", subcore_axis_name="s")

    @functools.partial(
        pl.kernel,
        out_type=jax.ShapeDtypeStruct((DW, NPAD), jnp.float32),
        mesh=mesh,
        compiler_params=_SC_PARAMS,
        scratch_types=[
            pltpu.VMEM((DPW,), jnp.int32),
            pltpu.VMEM((DPW,), jnp.float32),
            pltpu.VMEM((NPAD,), jnp.float32),
        ],
    )
    def body(dst_hbm, ew_hbm, zero_hbm, out_hbm, dst_v, ew_v, hist_v):
        c = lax.axis_index("c")
        s = lax.axis_index("s")
        w = s * 2 + c
        pltpu.sync_copy(dst_hbm.at[w], dst_v)
        pltpu.sync_copy(ew_hbm.at[w], ew_v)
        pltpu.sync_copy(zero_hbm, hist_v)

        def step(i, carry):
            d = dst_v[pl.ds(i * 16, 16)]
            e = ew_v[pl.ds(i * 16, 16)]
            plsc.addupdate_scatter(hist_v, [d], e)
            return carry

        lax.fori_loop(0, DCHUNK, step, 0)
        pltpu.sync_copy(hist_v, out_hbm.at[w])

    return body(dst_r, ew_r, zero_hist)


def _msg_call(xwp, src3, dst3, ew3):
    mesh = plsc.VectorSubcoreMesh(core_axis_name="c", subcore_axis_name="s")

    @functools.partial(
        pl.kernel,
        out_type=jax.ShapeDtypeStruct((2, NSLAB, NN, HALF), jnp.float32),
        mesh=mesh,
        compiler_params=_SC_PARAMS,
        scratch_types=[
            pltpu.VMEM((CHI, K), jnp.int32),
            pltpu.VMEM((CHI, K), jnp.int32),
            pltpu.VMEM((CHI, K), jnp.float32),
            pltpu.VMEM((K, HALF), jnp.float32),
            pltpu.VMEM((K, HALF), jnp.float32),
            pltpu.VMEM((K, HALF), jnp.float32),
            pltpu.VMEM((16, HALF), jnp.float32),
            pltpu.VMEM_SHARED((NN, HALF), jnp.float32),
            pltpu.SemaphoreType.DMA,
            pltpu.SemaphoreType.DMA,
            pltpu.SemaphoreType.DMA,
            pltpu.SemaphoreType.DMA,
            pltpu.SemaphoreType.DMA,
            pltpu.SemaphoreType.DMA,
            pltpu.SemaphoreType.DMA,
        ],
    )
    def body(xwp_hbm, src_hbm, dst_hbm, ew_hbm, out_hbm,
             src_v, dst_v, ew_v, r0, r1, r2, zbuf_v, acc_sh,
             g0, g1, g2, s0, s1, s2, zsem):
        c = lax.axis_index("c")
        s = lax.axis_index("s")
        base = s * STRIPE
        rows = (r0, r1, r2)
        gsem = (g0, g1, g2)
        ssem = (s0, s1, s2)

        zeros16 = jnp.zeros((16,), jnp.float32)

        def zrow(r, carry):
            for q in range(8):
                zbuf_v[r, pl.ds(q * 16, 16)] = zeros16
            return carry

        lax.fori_loop(0, 16, zrow, 0)

        def drain_scatter(i):
            pltpu.make_async_copy(
                xwp_hbm.at[0, 0, pl.ds(0, K)], rows[i], ssem[i]).wait()

        def scale(buf, cc):
            def scale_g(g, carry2):
                wv = ew_v[cc, pl.ds(g * 16, 16)]
                for rr in range(16):
                    w = wv[rr]
                    r = g * 16 + rr
                    for q in range(8):
                        sl = pl.ds(q * 16, 16)
                        buf[r, sl] = buf[r, sl] * w
                return carry2

            lax.fori_loop(0, K // 16, scale_g, 0)

        def slab(j, carry):
            def zloop(t, carry2):
                pltpu.async_copy(
                    zbuf_v, acc_sh.at[pl.ds(base + t * 16, 16)], zsem)
                return carry2

            lax.fori_loop(0, STRIPE // 16, zloop, 0)

            @pl.when(s == 15)
            def _():
                pltpu.async_copy(
                    zbuf_v, acc_sh.at[pl.ds(16 * STRIPE, 16)], zsem)

            def zdrain(t, carry2):
                pltpu.make_async_copy(
                    xwp_hbm.at[0, 0, pl.ds(0, 16)], zbuf_v, zsem).wait()
                return carry2

            lax.fori_loop(0, STRIPE // 16, zdrain, 0)

            @pl.when(s == 15)
            def _():
                pltpu.make_async_copy(
                    xwp_hbm.at[0, 0, pl.ds(0, 16)], zbuf_v, zsem).wait()

            plsc.subcore_barrier()

            def pair(p, carry2):
                is_stage = (p % PPB) == 0
                bo = p // PPB

                @pl.when(jnp.logical_and(is_stage, p > 0))
                def _():
                    for i in range(3):
                        drain_scatter(i)

                @pl.when(is_stage)
                def _():
                    da = pltpu.async_copy(src_hbm.at[s, bo], src_v, zsem)
                    db = pltpu.async_copy(dst_hbm.at[s, bo], dst_v, zsem)
                    dc = pltpu.async_copy(ew_hbm.at[s, bo], ew_v, zsem)
                    da.wait()
                    db.wait()
                    dc.wait()

                @pl.when(jnp.logical_and(jnp.logical_not(is_stage), p > 0))
                def _():
                    for i in range(3):
                        drain_scatter(i)

                q0 = (p % PPB) * 3
                gd = []
                for i in range(3):
                    gd.append(pltpu.async_copy(
                        xwp_hbm.at[c, j].at[src_v.at[q0 + i]],
                        rows[i], gsem[i]))
                for i in range(3):
                    gd[i].wait()
                    scale(rows[i], q0 + i)
                    pltpu.async_copy(
                        rows[i], acc_sh.at[dst_v.at[q0 + i]], ssem[i],
                        add=True)
                return carry2

            lax.fori_loop(0, NP, pair, 0)
            for i in range(3):
                drain_scatter(i)
            plsc.subcore_barrier()
            pltpu.sync_copy(acc_sh.at[pl.ds(base, STRIPE)],
                            out_hbm.at[c, j, pl.ds(base, STRIPE)])

            @pl.when(s == 15)
            def _():
                pltpu.sync_copy(
                    acc_sh.at[pl.ds(16 * STRIPE, NN - 16 * STRIPE)],
                    out_hbm.at[c, j, pl.ds(16 * STRIPE, NN - 16 * STRIPE)])

            plsc.subcore_barrier()
            return carry

        lax.fori_loop(0, NSLAB, slab, 0)

    return body(xwp, src3, dst3, ew3)


def _xw_body(x_ref, w_ref, deg_ref, out_ref):
    deg = jnp.sum(deg_ref[0], axis=0) + 1.0
    dinv = lax.rsqrt(deg)
    xw = jnp.dot(x_ref[0], w_ref[...], preferred_element_type=jnp.float32)
    out_ref[0, 0] = xw * dinv[:, None]


def _xw_call(x4, w, deg3):
    grid = (2, NSLAB, NBLK)
    return pl.pallas_call(
        _xw_body,
        grid=grid,
        in_specs=[
            pl.BlockSpec((1, ROW_BLK, FEAT), lambda c, j, n: (j, n, 0)),
            pl.BlockSpec((FEAT, HALF), lambda c, j, n: (0, c)),
            pl.BlockSpec((1, DW, ROW_BLK), lambda c, j, n: (n, 0, 0)),
        ],
        out_specs=pl.BlockSpec((1, 1, ROW_BLK, HALF), lambda c, j, n: (c, j, n, 0)),
        out_shape=jax.ShapeDtypeStruct((2, NSLAB, NN, HALF), jnp.float32),
    )(x4, w, deg3)


def _rnn_body(s_ref, xwp_ref, deg_ref, b_ref, wih_t_ref, whh_t_ref,
              bih_ref, bhh_ref, out_ref):
    deg = jnp.sum(deg_ref[0], axis=0) + 1.0
    dinv = lax.rsqrt(deg)[:, None]
    b = b_ref[...]
    bias = bih_ref[...] + bhh_ref[...]
    wih_t = wih_t_ref[...]
    whh_t = whh_t_ref[...]

    def conv(j):
        lo = dinv * (s_ref[0, j] + xwp_ref[0, j])
        hi = dinv * (s_ref[1, j] + xwp_ref[1, j])
        return jnp.concatenate([lo, hi], axis=1) + b

    h = jnp.zeros((ROW_BLK, FEAT), dtype=jnp.float32)
    for k, j in enumerate((0, 2, 1, 3)):
        h = jnp.tanh(
            jnp.dot(conv(j), wih_t, preferred_element_type=jnp.float32)
            + jnp.dot(h, whh_t, preferred_element_type=jnp.float32)
            + bias
        )
        if k == 2:
            out_ref[0] = h
    out_ref[1] = h


def _rnn_call(s, xwp, deg3, b, wih_t, whh_t, bih, bhh):
    grid = (NBLK,)
    return pl.pallas_call(
        _rnn_body,
        grid=grid,
        in_specs=[
            pl.BlockSpec((2, NSLAB, ROW_BLK, HALF), lambda n: (0, 0, n, 0)),
            pl.BlockSpec((2, NSLAB, ROW_BLK, HALF), lambda n: (0, 0, n, 0)),
            pl.BlockSpec((1, DW, ROW_BLK), lambda n: (n, 0, 0)),
            pl.BlockSpec((FEAT,), lambda n: (0,)),
            pl.BlockSpec((FEAT, FEAT), lambda n: (0, 0)),
            pl.BlockSpec((FEAT, FEAT), lambda n: (0, 0)),
            pl.BlockSpec((FEAT,), lambda n: (0,)),
            pl.BlockSpec((FEAT,), lambda n: (0,)),
        ],
        out_specs=pl.BlockSpec((2, ROW_BLK, FEAT), lambda n: (0, n, 0)),
        out_shape=jax.ShapeDtypeStruct((2, NN, FEAT), jnp.float32),
    )(s, xwp, deg3, b, wih_t, whh_t, bih, bhh)


def kernel(x_in, edge_index, edge_weight, W, b, W_ih, W_hh, b_ih, b_hh):
    B, T, N, F = x_in.shape
    src = edge_index[0]
    dst = edge_index[1]

    dpad = DW * DPW - EDG
    dst_r = jnp.pad(dst, (0, dpad)).reshape(DW, DPW)
    ew_r = jnp.pad(edge_weight, (0, dpad)).reshape(DW, DPW)
    zero_hist = jnp.zeros((NPAD,), jnp.float32)
    deg_parts = _deg_call(dst_r, ew_r, zero_hist)
    deg3 = (deg_parts[:, :NN]
            .reshape(DW, NBLK, ROW_BLK).transpose(1, 0, 2))

    x4 = x_in.reshape(B * T, N, F)
    xwp = _xw_call(x4, W, deg3)

    mpad = 16 * PERT - EDG
    src3 = jnp.pad(src, (0, mpad)).reshape(16, CHO, CHI, K)
    dst3 = jnp.pad(dst, (0, mpad)).reshape(16, CHO, CHI, K)
    ew3 = jnp.pad(edge_weight, (0, mpad)).reshape(16, CHO, CHI, K)
    s = _msg_call(xwp, src3, dst3, ew3)

    return _rnn_call(s, xwp, deg3, b, W_ih.T, W_hh.T, b_ih, b_hh)
